# baseline (speedup 1.0000x reference)
import functools

import jax
import jax.numpy as jnp
from jax import lax
from jax.experimental import pallas as pl
from jax.experimental.pallas import tpu as pltpu
from jax.experimental.pallas import tpu_sc as plsc

_B = 16384
_D = 128
_NCORES = 2
_NSUB = 16
_NW = _NCORES * _NSUB

_NCHUNK = 2
_NB = _B // _NCHUNK
_BM = 2048


def _make_sc_body(nb):
    bpw = nb // _NW

    def body(uid_hbm, qid_hbm, user_t, qdiff_t, qtab_t, qdisc_t,
             u_out, d_out, m_out, disc_out,
             uid_v, qid_v, rows_v, disc_v, sem, dsem):
        wid = lax.axis_index("s") * _NCORES + lax.axis_index("c")
        base = wid * bpw
        pltpu.sync_copy(uid_hbm.at[pl.ds(base, bpw)], uid_v)
        pltpu.sync_copy(qid_hbm.at[pl.ds(base, bpw)], qid_v)
        gd = pltpu.async_copy(qdisc_t.at[qid_v], disc_v, dsem)
        pltpu.async_copy(user_t.at[uid_v], rows_v, sem).wait()
        pltpu.sync_copy(rows_v, u_out.at[pl.ds(base, bpw)])
        pltpu.async_copy(qdiff_t.at[qid_v], rows_v, sem).wait()
        pltpu.sync_copy(rows_v, d_out.at[pl.ds(base, bpw)])
        pltpu.async_copy(qtab_t.at[qid_v], rows_v, sem).wait()
        pltpu.sync_copy(rows_v, m_out.at[pl.ds(base, bpw)])
        gd.wait()
        pltpu.sync_copy(disc_v, disc_out.at[pl.ds(base, bpw)])

    return body


@functools.cache
def _sc_gather(nb):
    bpw = nb // _NW
    return pl.kernel(
        _make_sc_body(nb),
        out_type=[
            jax.ShapeDtypeStruct((nb, _D), jnp.float32),
            jax.ShapeDtypeStruct((nb, _D), jnp.float32),
            jax.ShapeDtypeStruct((nb, _D), jnp.float32),
            jax.ShapeDtypeStruct((nb,), jnp.float32),
        ],
        mesh=plsc.VectorSubcoreMesh(core_axis_name="c", subcore_axis_name="s",
                                    num_cores=_NCORES, num_subcores=_NSUB),
        scratch_types=[
            pltpu.VMEM((bpw,), jnp.int32),
            pltpu.VMEM((bpw,), jnp.int32),
            pltpu.VMEM((bpw, _D), jnp.float32),
            pltpu.VMEM((bpw,), jnp.float32),
            pltpu.SemaphoreType.DMA,
            pltpu.SemaphoreType.DMA,
        ],
    )


def _sigmoid(x):
    return 0.5 * jnp.tanh(0.5 * x) + 0.5


def _mlp_body(u_ref, d_ref, m_ref, disc_ref, w1_ref, b1_ref, w2_ref, b2_ref,
              w3t_ref, b3_ref, out_ref):
    u = _sigmoid(u_ref[...])
    d = _sigmoid(d_ref[...])
    disc = _sigmoid(disc_ref[...].T) * 10.0
    x = disc * (u - d) * m_ref[...]
    h = _sigmoid(
        jnp.dot(x, w1_ref[...], preferred_element_type=jnp.float32) + b1_ref[...])
    h = _sigmoid(
        jnp.dot(h, w2_ref[...], preferred_element_type=jnp.float32) + b2_ref[...])
    o = jnp.sum(h * w3t_ref[...], axis=-1, keepdims=True) + b3_ref[...]
    out_ref[...] = _sigmoid(o).T


def _tc_mlp(u_rows, d_rows, m_rows, disc, W1, b1r, W2, b2r, w3t, b3r):
    nb = u_rows.shape[0]
    grid = nb // _BM
    row_spec = pl.BlockSpec((_BM, _D), lambda i: (i, 0))
    rowvec_spec = pl.BlockSpec((1, _BM), lambda i: (0, i))
    full = lambda shape: pl.BlockSpec(shape, lambda i: (0,) * len(shape))
    return pl.pallas_call(
        _mlp_body,
        grid=(grid,),
        in_specs=[
            row_spec, row_spec, row_spec, rowvec_spec,
            full((128, 512)), full((1, 512)),
            full((512, 256)), full((1, 256)),
            full((1, 256)), full((1, 1)),
        ],
        out_specs=rowvec_spec,
        out_shape=jax.ShapeDtypeStruct((1, nb), jnp.float32),
        compiler_params=pltpu.CompilerParams(
            dimension_semantics=("arbitrary",)),
    )(u_rows, d_rows, m_rows, disc.reshape(1, nb), W1, b1r, W2, b2r, w3t, b3r)


@jax.jit
def _ncd_forward(uid, qid, q_table, user_table, q_diff_table, q_disc_table,
                 W1, b1, W2, b2, W3, b3):
    qdisc_1d = q_disc_table.reshape(-1)
    b1r = b1.reshape(1, -1)
    b2r = b2.reshape(1, -1)
    w3t = W3.reshape(1, -1)
    b3r = b3.reshape(1, 1)
    gathered = []
    for c in range(_NCHUNK):
        sl = slice(c * _NB, (c + 1) * _NB)
        gathered.append(_sc_gather(_NB)(
            uid[sl], qid[sl], user_table, q_diff_table, q_table, qdisc_1d))
    outs = []
    for c in range(_NCHUNK):
        u_rows, d_rows, m_rows, disc = gathered[c]
        outs.append(_tc_mlp(u_rows, d_rows, m_rows, disc,
                            W1, b1r, W2, b2r, w3t, b3r))
    return jnp.concatenate(outs, axis=1).reshape(-1)


def kernel(user_id, question_id, q_table, user_table, q_diff_table,
           q_disc_table, W1, b1, W2, b2, W3, b3):
    uid = user_id.astype(jnp.int32)
    qid = question_id.astype(jnp.int32)
    return _ncd_forward(uid, qid, q_table, user_table, q_diff_table,
                        q_disc_table, W1, b1, W2, b2, W3, b3)
